# baseline (speedup 1.0000x reference)
import functools
import math

import jax
import jax.numpy as jnp
from jax import lax
from jax.experimental import pallas as pl
from jax.experimental.pallas import tpu as pltpu
from jax.experimental.pallas import tpu_sc as plsc

N = 10000
E = 320000
D = 128
NT = 4
ET = 8

NC = 2
NS = 16
NW = NC * NS
CE = E // NW
G = 80
NB = CE // G
LG = G // 16
NPAD = 10240
SL = NPAD // NS
RH = 624

BN = 1000


def _qvke_body(x_ref, nt_ref, wq_ref, wk_ref, wv_ref, we_ref, musc_ref,
               q_ref, v_ref, ke_ref):
    xb = x_ref[...]
    nt = nt_ref[...]
    k = jnp.zeros_like(xb)
    q = jnp.zeros_like(xb)
    v = jnp.zeros_like(xb)
    for t in range(NT):
        m = nt == t
        q_t = jnp.dot(xb, wq_ref[t], preferred_element_type=jnp.float32)
        k_t = jnp.dot(xb, wk_ref[t], preferred_element_type=jnp.float32)
        v_t = jnp.dot(xb, wv_ref[t], preferred_element_type=jnp.float32)
        q = q + jnp.where(m, q_t, 0.0)
        v = v + jnp.where(m, v_t, 0.0)
        k = k + jnp.where(m, k_t, 0.0)
    q_ref[...] = q
    v_ref[...] = v
    for t in range(ET):
        ke_ref[t] = (
            jnp.dot(k, we_ref[t], preferred_element_type=jnp.float32)
            * musc_ref[t]
        )


def _qvke(x, nt2d, w_q, w_k, w_v, w_e, musc):
    return pl.pallas_call(
        _qvke_body,
        grid=(N // BN,),
        in_specs=[
            pl.BlockSpec((BN, D), lambda i: (i, 0)),
            pl.BlockSpec((BN, 1), lambda i: (i, 0)),
            pl.BlockSpec((NT, D, D), lambda i: (0, 0, 0)),
            pl.BlockSpec((NT, D, D), lambda i: (0, 0, 0)),
            pl.BlockSpec((NT, D, D), lambda i: (0, 0, 0)),
            pl.BlockSpec((ET, D, D), lambda i: (0, 0, 0)),
            pl.BlockSpec(memory_space=pltpu.SMEM),
        ],
        out_specs=[
            pl.BlockSpec((BN, D), lambda i: (i, 0)),
            pl.BlockSpec((BN, D), lambda i: (i, 0)),
            pl.BlockSpec((ET, BN, D), lambda i: (0, i, 0)),
        ],
        out_shape=[
            jax.ShapeDtypeStruct((N, D), jnp.float32),
            jax.ShapeDtypeStruct((N, D), jnp.float32),
            jax.ShapeDtypeStruct((ET, N, D), jnp.float32),
        ],
    )(x, nt2d, w_q, w_k, w_v, w_e, musc)


_SC_MESH = plsc.VectorSubcoreMesh(
    core_axis_name="c", subcore_axis_name="s", num_cores=NC, num_subcores=NS
)


@functools.partial(
    pl.kernel,
    out_type=[
        jax.ShapeDtypeStruct((E,), jnp.float32),
        jax.ShapeDtypeStruct((NW * NPAD,), jnp.float32),
    ],
    mesh=_SC_MESH,
    scratch_types=[
        pltpu.VMEM((G,), jnp.int32),
        pltpu.VMEM((G,), jnp.int32),
        pltpu.VMEM((G,), jnp.int32),
        pltpu.VMEM((G,), jnp.int32),
        pltpu.VMEM((G,), jnp.int32),
        pltpu.VMEM((G,), jnp.int32),
        pltpu.VMEM((G,), jnp.int32),
        pltpu.VMEM((G,), jnp.int32),
        pltpu.VMEM((G, D), jnp.float32),
        pltpu.VMEM((G, D), jnp.float32),
        pltpu.VMEM((G, D), jnp.float32),
        pltpu.VMEM((G, D), jnp.float32),
        pltpu.VMEM((G,), jnp.float32),
        pltpu.VMEM((G,), jnp.float32),
        pltpu.VMEM((NPAD,), jnp.float32),
        pltpu.SemaphoreType.DMA,
        pltpu.SemaphoreType.DMA,
        pltpu.SemaphoreType.DMA,
        pltpu.SemaphoreType.DMA,
        pltpu.SemaphoreType.DMA,
        pltpu.SemaphoreType.DMA,
    ],
    compiler_params=pltpu.CompilerParams(needs_layout_passes=False),
)
def _scores_kernel(q_hbm, ke_hbm, src_hbm, dst_hbm, et_hbm,
                   ex_hbm, ssum_hbm,
                   dst0, dst1, src0, src1, et0, et1, ki0, ki1,
                   qr0, qr1, kr0, kr1, exb0, exb1, ssum_v,
                   semi0, semi1, semg0, semg1, semx0, semx1):
    c = lax.axis_index("c")
    s = lax.axis_index("s")
    wid = s * NC + c
    base = wid * CE

    bufs = (
        dict(dst=dst0, src=src0, et=et0, ki=ki0, qr=qr0, kr=kr0, exb=exb0,
             semi=semi0, semg=semg0, semx=semx0),
        dict(dst=dst1, src=src1, et=et1, ki=ki1, qr=qr1, kr=kr1, exb=exb1,
             semi=semi1, semg=semg1, semx=semx1),
    )
    def wait_ex(B):
        pltpu.make_async_copy(
            B["exb"], ex_hbm.at[pl.ds(base, G)], B["semx"]
        ).wait()

    def fire_idx(b, B):
        off = base + b * G
        pltpu.async_copy(dst_hbm.at[pl.ds(off, G)], B["dst"], B["semi"])
        pltpu.async_copy(src_hbm.at[pl.ds(off, G)], B["src"], B["semi"])
        pltpu.async_copy(et_hbm.at[pl.ds(off, G)], B["et"], B["semi"])

    def wait_idx(B):
        z = dst_hbm.at[pl.ds(0, G)]
        pltpu.make_async_copy(z, B["dst"], B["semi"]).wait()
        pltpu.make_async_copy(z, B["src"], B["semi"]).wait()
        pltpu.make_async_copy(z, B["et"], B["semi"]).wait()

    def kidx(B):
        def kidx_body(g, _):
            ix = pl.ds(g * 16, 16)
            B["ki"][ix] = B["et"][ix] * N + B["src"][ix]
            return 0

        lax.fori_loop(0, LG, kidx_body, 0)

    def fire_gather(B):
        pltpu.async_copy(q_hbm.at[B["dst"]], B["qr"], B["semg"])
        pltpu.async_copy(ke_hbm.at[B["ki"]], B["kr"], B["semg"])

    def wait_gather(B):
        pltpu.make_async_copy(q_hbm.at[B["dst"]], B["qr"], B["semg"]).wait()
        pltpu.make_async_copy(ke_hbm.at[B["ki"]], B["kr"], B["semg"]).wait()

    lanes = lax.iota(jnp.int32, 16)

    def compute(b, B):
        qrows, kerows, dstb, exb = B["qr"], B["kr"], B["dst"], B["exb"]

        @pl.when(b >= 2)
        def _():
            wait_ex(B)

        def grp_body(g, _):
            rows = jnp.full((16,), g * 16, jnp.int32) + lanes

            def dot_body(d, carry):
                dc, acc = carry
                vq = plsc.load_gather(qrows, [rows, dc])
                vk = plsc.load_gather(kerows, [rows, dc])
                return (dc + 1, acc + vq * vk)

            _, acc16 = lax.fori_loop(
                0,
                D,
                dot_body,
                (jnp.zeros((16,), jnp.int32), jnp.zeros((16,), jnp.float32)),
            )
            ex = jnp.exp(acc16)
            exb[pl.ds(g * 16, 16)] = ex
            dv = dstb[pl.ds(g * 16, 16)]
            plsc.addupdate_scatter(ssum_v, [dv], ex)
            return 0

        lax.fori_loop(0, LG, grp_body, 0)
        pltpu.async_copy(exb, ex_hbm.at[pl.ds(base + b * G, G)], B["semx"])

    def zero_body(i, _):
        ssum_v[pl.ds(i * 16, 16)] = jnp.zeros((16,), jnp.float32)
        return 0

    lax.fori_loop(0, NPAD // 16, zero_body, 0)

    fire_idx(0, bufs[0])
    wait_idx(bufs[0])
    kidx(bufs[0])
    fire_gather(bufs[0])
    fire_idx(1, bufs[1])

    def pair_body(t, _):
        b0 = 2 * t
        wait_gather(bufs[0])
        wait_idx(bufs[1])
        kidx(bufs[1])
        fire_gather(bufs[1])
        compute(b0, bufs[0])
        fire_idx(b0 + 2, bufs[0])

        wait_gather(bufs[1])
        wait_idx(bufs[0])
        kidx(bufs[0])
        fire_gather(bufs[0])
        compute(b0 + 1, bufs[1])

        @pl.when(b0 + 3 < NB)
        def _():
            fire_idx(b0 + 3, bufs[1])

        return 0

    lax.fori_loop(0, (NB - 1) // 2, pair_body, 0)
    wait_gather(bufs[0])
    compute(NB - 1, bufs[0])
    wait_ex(bufs[0])
    wait_ex(bufs[1])
    pltpu.sync_copy(ssum_v, ssum_hbm.at[pl.ds(wid * NPAD, NPAD)])


@functools.partial(
    pl.kernel,
    out_type=jax.ShapeDtypeStruct((NC, NPAD, D), jnp.float32),
    mesh=_SC_MESH,
    scratch_types=[
        pltpu.VMEM((G,), jnp.int32),
        pltpu.VMEM((G,), jnp.int32),
        pltpu.VMEM((G,), jnp.int32),
        pltpu.VMEM((G,), jnp.int32),
        pltpu.VMEM((G,), jnp.float32),
        pltpu.VMEM((G,), jnp.float32),
        pltpu.VMEM((G, D), jnp.float32),
        pltpu.VMEM((G, D), jnp.float32),
        pltpu.VMEM((G,), jnp.int32),
        pltpu.VMEM((G,), jnp.int32),
        pltpu.VMEM_SHARED((N, D), jnp.float32),
        pltpu.SemaphoreType.DMA,
        pltpu.SemaphoreType.DMA,
        pltpu.SemaphoreType.DMA,
        pltpu.SemaphoreType.DMA,
        pltpu.SemaphoreType.DMA,
        pltpu.SemaphoreType.DMA,
    ],
    compiler_params=pltpu.CompilerParams(needs_layout_passes=False),
)
def _agg_kernel(v_hbm, src_hbm, dst_hbm, ex_hbm, outp_hbm,
                dst0, dst1, src0, src1, ex0, ex1, vr0, vr1, dsc0, dsc1,
                out_sh,
                semi0, semi1, semv0, semv1, sems0, sems1):
    c = lax.axis_index("c")
    s = lax.axis_index("s")
    wid = s * NC + c
    base = wid * CE
    vrows = vr0

    def vz_body(r, _):
        for j in range(D // 16):
            vrows[r, pl.ds(j * 16, 16)] = jnp.zeros((16,), jnp.float32)
        return 0

    lax.fori_loop(0, G, vz_body, 0)
    row0 = s * RH
    for i in range(RH // G):
        pltpu.sync_copy(vrows, out_sh.at[pl.ds(row0 + i * G, G)])
    rem = RH % G
    pltpu.sync_copy(
        vrows.at[pl.ds(0, rem)],
        out_sh.at[pl.ds(row0 + (RH // G) * G, rem)],
    )

    @pl.when(s == NS - 1)
    def _zero_tail():
        pltpu.sync_copy(
            vrows.at[pl.ds(0, N - NS * RH)],
            out_sh.at[pl.ds(NS * RH, N - NS * RH)],
        )

    plsc.subcore_barrier()

    bufs = (
        dict(dst=dst0, src=src0, ex=ex0, vr=vr0, dsc=dsc0,
             semi=semi0, semv=semv0, sems=sems0),
        dict(dst=dst1, src=src1, ex=ex1, vr=vr1, dsc=dsc1,
             semi=semi1, semv=semv1, sems=sems1),
    )
    def wait_scat(B):
        pltpu.make_async_copy(
            B["vr"], out_sh.at[B["dsc"]], B["sems"]
        ).wait()

    def fire_idx(b, B):
        off = base + b * G
        pltpu.async_copy(dst_hbm.at[pl.ds(off, G)], B["dst"], B["semi"])
        pltpu.async_copy(src_hbm.at[pl.ds(off, G)], B["src"], B["semi"])
        pltpu.async_copy(ex_hbm.at[pl.ds(off, G)], B["ex"], B["semi"])

    def wait_idx(B):
        zi = dst_hbm.at[pl.ds(0, G)]
        zf = ex_hbm.at[pl.ds(0, G)]
        pltpu.make_async_copy(zi, B["dst"], B["semi"]).wait()
        pltpu.make_async_copy(zi, B["src"], B["semi"]).wait()
        pltpu.make_async_copy(zf, B["ex"], B["semi"]).wait()

    def fire_v(B):
        pltpu.async_copy(v_hbm.at[B["src"]], B["vr"], B["semv"])

    def wait_v(B):
        pltpu.make_async_copy(v_hbm.at[B["src"]], B["vr"], B["semv"]).wait()

    def compute(B):
        dstb, exb, vr, dsc = B["dst"], B["ex"], B["vr"], B["dsc"]

        def cp_body(g, _):
            ix = pl.ds(g * 16, 16)
            dsc[ix] = dstb[ix]
            return 0

        lax.fori_loop(0, LG, cp_body, 0)

        def scale_body(g, _):
            gbase = g * 16
            for i in range(16):
                e = gbase + i
                wv = plsc.load_gather(exb, [jnp.full((16,), e, jnp.int32)])
                for j in range(D // 16):
                    ix = pl.ds(j * 16, 16)
                    vr[e, ix] = vr[e, ix] * wv
            return 0

        lax.fori_loop(0, LG, scale_body, 0)
        pltpu.async_copy(vr, out_sh.at[dsc], B["sems"], add=True)

    fire_idx(0, bufs[0])
    wait_idx(bufs[0])
    fire_v(bufs[0])
    fire_idx(1, bufs[1])

    def pair_body(t, _):
        b0 = 2 * t
        wait_v(bufs[0])
        wait_idx(bufs[1])

        @pl.when(b0 >= 2)
        def _():
            wait_scat(bufs[1])

        fire_v(bufs[1])
        compute(bufs[0])
        fire_idx(b0 + 2, bufs[0])

        wait_v(bufs[1])
        wait_idx(bufs[0])
        wait_scat(bufs[0])
        fire_v(bufs[0])
        compute(bufs[1])

        @pl.when(b0 + 3 < NB)
        def _():
            fire_idx(b0 + 3, bufs[1])

        return 0

    lax.fori_loop(0, (NB - 1) // 2, pair_body, 0)
    wait_v(bufs[0])
    compute(bufs[0])
    wait_scat(bufs[0])
    wait_scat(bufs[1])
    plsc.subcore_barrier()

    rem = RH % G
    for i in range(RH // G):
        r0 = s * RH + i * G
        pltpu.sync_copy(out_sh.at[pl.ds(r0, G)], vrows)
        pltpu.sync_copy(vrows, outp_hbm.at[c, pl.ds(r0, G)])
    r0 = s * RH + (RH // G) * G
    pltpu.sync_copy(out_sh.at[pl.ds(r0, rem)], vrows.at[pl.ds(0, rem)])
    pltpu.sync_copy(vrows.at[pl.ds(0, rem)], outp_hbm.at[c, pl.ds(r0, rem)])

    @pl.when(s == NS - 1)
    def _write_tail():
        nt_ = N - NS * RH
        pltpu.sync_copy(
            out_sh.at[pl.ds(NS * RH, nt_)], vrows.at[pl.ds(0, nt_)]
        )
        pltpu.sync_copy(
            vrows.at[pl.ds(0, nt_)], outp_hbm.at[c, pl.ds(NS * RH, nt_)]
        )


BN2 = 1024


def _norm_body(p_ref, ss_ref, o_ref):
    s2 = jnp.sum(ss_ref[...], axis=0).reshape(BN2, 1)
    o_ref[...] = (p_ref[0] + p_ref[1]) / (s2 + 1e-10)


def _normalize(outp, ssump):
    return pl.pallas_call(
        _norm_body,
        grid=(NPAD // BN2,),
        in_specs=[
            pl.BlockSpec((NC, BN2, D), lambda i: (0, i, 0)),
            pl.BlockSpec((NW, BN2), lambda i: (0, i)),
        ],
        out_specs=pl.BlockSpec((BN2, D), lambda i: (i, 0)),
        out_shape=jax.ShapeDtypeStruct((NPAD, D), jnp.float32),
    )(outp, ssump)


def kernel(x, edge_index, edge_type, node_type, W_Q, W_K, W_V, W_edge, mu):
    src = edge_index[0]
    dst = edge_index[1]
    nt2d = node_type.reshape(N, 1)
    musc = (mu / math.sqrt(D)).astype(jnp.float32)
    q, v, ke = _qvke(x, nt2d, W_Q, W_K, W_V, W_edge, musc)
    ke2 = ke.reshape(ET * N, D)
    ex, ssump = _scores_kernel(q, ke2, src, dst, edge_type)
    outp = _agg_kernel(v, src, dst, ex)
    return _normalize(outp, ssump.reshape(NW, NPAD))[:N]


# speedup vs baseline: 3.3686x; 3.3686x over previous
import functools
import math

import jax
import jax.numpy as jnp
from jax import lax
from jax.experimental import pallas as pl
from jax.experimental.pallas import tpu as pltpu
from jax.experimental.pallas import tpu_sc as plsc

N = 10000
E = 320000
D = 128
NT = 4
ET = 8

NC = 2
NS = 16
NW = NC * NS
CE = E // NW
G = 80
NB = CE // G
LG = G // 16
NPAD = 10240
SL = NPAD // NS
RH = 624

BN = 1000


def _qvke_body(x_ref, nt_ref, wq_ref, wk_ref, wv_ref, we_ref, musc_ref,
               q_ref, v_ref, ke_ref):
    xb = x_ref[...]
    nt = nt_ref[...]
    k = jnp.zeros_like(xb)
    q = jnp.zeros_like(xb)
    v = jnp.zeros_like(xb)
    for t in range(NT):
        m = nt == t
        q_t = jnp.dot(xb, wq_ref[t], preferred_element_type=jnp.float32)
        k_t = jnp.dot(xb, wk_ref[t], preferred_element_type=jnp.float32)
        v_t = jnp.dot(xb, wv_ref[t], preferred_element_type=jnp.float32)
        q = q + jnp.where(m, q_t, 0.0)
        v = v + jnp.where(m, v_t, 0.0)
        k = k + jnp.where(m, k_t, 0.0)
    q_ref[...] = q
    v_ref[...] = v
    for t in range(ET):
        ke_ref[t] = (
            jnp.dot(k, we_ref[t], preferred_element_type=jnp.float32)
            * musc_ref[t]
        )


def _qvke(x, nt2d, w_q, w_k, w_v, w_e, musc):
    return pl.pallas_call(
        _qvke_body,
        grid=(N // BN,),
        in_specs=[
            pl.BlockSpec((BN, D), lambda i: (i, 0)),
            pl.BlockSpec((BN, 1), lambda i: (i, 0)),
            pl.BlockSpec((NT, D, D), lambda i: (0, 0, 0)),
            pl.BlockSpec((NT, D, D), lambda i: (0, 0, 0)),
            pl.BlockSpec((NT, D, D), lambda i: (0, 0, 0)),
            pl.BlockSpec((ET, D, D), lambda i: (0, 0, 0)),
            pl.BlockSpec(memory_space=pltpu.SMEM),
        ],
        out_specs=[
            pl.BlockSpec((BN, D), lambda i: (i, 0)),
            pl.BlockSpec((BN, D), lambda i: (i, 0)),
            pl.BlockSpec((ET, BN, D), lambda i: (0, i, 0)),
        ],
        out_shape=[
            jax.ShapeDtypeStruct((N, D), jnp.float32),
            jax.ShapeDtypeStruct((N, D), jnp.float32),
            jax.ShapeDtypeStruct((ET, N, D), jnp.float32),
        ],
    )(x, nt2d, w_q, w_k, w_v, w_e, musc)


_SC_MESH = plsc.VectorSubcoreMesh(
    core_axis_name="c", subcore_axis_name="s", num_cores=NC, num_subcores=NS
)


@functools.partial(
    pl.kernel,
    out_type=[
        jax.ShapeDtypeStruct((E,), jnp.float32),
        jax.ShapeDtypeStruct((NW * NPAD,), jnp.float32),
    ],
    mesh=_SC_MESH,
    scratch_types=[
        pltpu.VMEM((G,), jnp.int32),
        pltpu.VMEM((G,), jnp.int32),
        pltpu.VMEM((G,), jnp.int32),
        pltpu.VMEM((G,), jnp.int32),
        pltpu.VMEM((G,), jnp.int32),
        pltpu.VMEM((G,), jnp.int32),
        pltpu.VMEM((G,), jnp.int32),
        pltpu.VMEM((G,), jnp.int32),
        pltpu.VMEM((G, D), jnp.float32),
        pltpu.VMEM((G, D), jnp.float32),
        pltpu.VMEM((G, D), jnp.float32),
        pltpu.VMEM((G, D), jnp.float32),
        pltpu.VMEM((G,), jnp.float32),
        pltpu.VMEM((G,), jnp.float32),
        pltpu.VMEM((NPAD,), jnp.float32),
        pltpu.SemaphoreType.DMA,
        pltpu.SemaphoreType.DMA,
        pltpu.SemaphoreType.DMA,
        pltpu.SemaphoreType.DMA,
        pltpu.SemaphoreType.DMA,
        pltpu.SemaphoreType.DMA,
    ],
    compiler_params=pltpu.CompilerParams(needs_layout_passes=False),
)
def _scores_kernel(q_hbm, ke_hbm, src_hbm, dst_hbm, et_hbm,
                   ex_hbm, ssum_hbm,
                   dst0, dst1, src0, src1, et0, et1, ki0, ki1,
                   qr0, qr1, kr0, kr1, exb0, exb1, ssum_v,
                   semi0, semi1, semg0, semg1, semx0, semx1):
    c = lax.axis_index("c")
    s = lax.axis_index("s")
    wid = s * NC + c
    base = wid * CE

    bufs = (
        dict(dst=dst0, src=src0, et=et0, ki=ki0, qr=qr0, kr=kr0, exb=exb0,
             semi=semi0, semg=semg0, semx=semx0),
        dict(dst=dst1, src=src1, et=et1, ki=ki1, qr=qr1, kr=kr1, exb=exb1,
             semi=semi1, semg=semg1, semx=semx1),
    )
    def wait_ex(B):
        pltpu.make_async_copy(
            B["exb"], ex_hbm.at[pl.ds(base, G)], B["semx"]
        ).wait()

    def fire_idx(b, B):
        off = base + b * G
        pltpu.async_copy(dst_hbm.at[pl.ds(off, G)], B["dst"], B["semi"])
        pltpu.async_copy(src_hbm.at[pl.ds(off, G)], B["src"], B["semi"])
        pltpu.async_copy(et_hbm.at[pl.ds(off, G)], B["et"], B["semi"])

    def wait_idx(B):
        z = dst_hbm.at[pl.ds(0, G)]
        pltpu.make_async_copy(z, B["dst"], B["semi"]).wait()
        pltpu.make_async_copy(z, B["src"], B["semi"]).wait()
        pltpu.make_async_copy(z, B["et"], B["semi"]).wait()

    def kidx(B):
        def kidx_body(g, _):
            ix = pl.ds(g * 16, 16)
            B["ki"][ix] = B["et"][ix] * N + B["src"][ix]
            return 0

        lax.fori_loop(0, LG, kidx_body, 0)

    def fire_gather(B):
        pltpu.async_copy(q_hbm.at[B["dst"]], B["qr"], B["semg"])
        pltpu.async_copy(ke_hbm.at[B["ki"]], B["kr"], B["semg"])

    def wait_gather(B):
        pltpu.make_async_copy(q_hbm.at[B["dst"]], B["qr"], B["semg"]).wait()
        pltpu.make_async_copy(ke_hbm.at[B["ki"]], B["kr"], B["semg"]).wait()

    lanes = lax.iota(jnp.int32, 16)

    def compute(b, B):
        qrows, kerows, dstb, exb = B["qr"], B["kr"], B["dst"], B["exb"]

        @pl.when(b >= 2)
        def _():
            wait_ex(B)

        def grp_body(g, _):
            def edge_body(i, grp):
                e = g * 16 + i
                acc = qrows[e, pl.ds(0, 16)] * kerows[e, pl.ds(0, 16)]
                for j in range(1, D // 16):
                    ix = pl.ds(j * 16, 16)
                    acc = acc + qrows[e, ix] * kerows[e, ix]
                total = jnp.sum(acc)
                return jnp.where(lanes == i, total, grp)

            acc16 = lax.fori_loop(
                0, 16, edge_body, jnp.zeros((16,), jnp.float32)
            )
            ex = jnp.exp(acc16)
            exb[pl.ds(g * 16, 16)] = ex
            dv = dstb[pl.ds(g * 16, 16)]
            plsc.addupdate_scatter(ssum_v, [dv], ex)
            return 0

        lax.fori_loop(0, LG, grp_body, 0)
        pltpu.async_copy(exb, ex_hbm.at[pl.ds(base + b * G, G)], B["semx"])

    def zero_body(i, _):
        ssum_v[pl.ds(i * 16, 16)] = jnp.zeros((16,), jnp.float32)
        return 0

    lax.fori_loop(0, NPAD // 16, zero_body, 0)

    fire_idx(0, bufs[0])
    wait_idx(bufs[0])
    kidx(bufs[0])
    fire_gather(bufs[0])
    fire_idx(1, bufs[1])

    def pair_body(t, _):
        b0 = 2 * t
        wait_gather(bufs[0])
        wait_idx(bufs[1])
        kidx(bufs[1])
        fire_gather(bufs[1])
        compute(b0, bufs[0])
        fire_idx(b0 + 2, bufs[0])

        wait_gather(bufs[1])
        wait_idx(bufs[0])
        kidx(bufs[0])
        fire_gather(bufs[0])
        compute(b0 + 1, bufs[1])

        @pl.when(b0 + 3 < NB)
        def _():
            fire_idx(b0 + 3, bufs[1])

        return 0

    lax.fori_loop(0, (NB - 1) // 2, pair_body, 0)
    wait_gather(bufs[0])
    compute(NB - 1, bufs[0])
    wait_ex(bufs[0])
    wait_ex(bufs[1])
    pltpu.sync_copy(ssum_v, ssum_hbm.at[pl.ds(wid * NPAD, NPAD)])


@functools.partial(
    pl.kernel,
    out_type=jax.ShapeDtypeStruct((NC, NPAD, D), jnp.float32),
    mesh=_SC_MESH,
    scratch_types=[
        pltpu.VMEM((G,), jnp.int32),
        pltpu.VMEM((G,), jnp.int32),
        pltpu.VMEM((G,), jnp.int32),
        pltpu.VMEM((G,), jnp.int32),
        pltpu.VMEM((G,), jnp.float32),
        pltpu.VMEM((G,), jnp.float32),
        pltpu.VMEM((G, D), jnp.float32),
        pltpu.VMEM((G, D), jnp.float32),
        pltpu.VMEM((G,), jnp.int32),
        pltpu.VMEM((G,), jnp.int32),
        pltpu.VMEM_SHARED((N, D), jnp.float32),
        pltpu.SemaphoreType.DMA,
        pltpu.SemaphoreType.DMA,
        pltpu.SemaphoreType.DMA,
        pltpu.SemaphoreType.DMA,
        pltpu.SemaphoreType.DMA,
        pltpu.SemaphoreType.DMA,
    ],
    compiler_params=pltpu.CompilerParams(needs_layout_passes=False),
)
def _agg_kernel(v_hbm, src_hbm, dst_hbm, ex_hbm, outp_hbm,
                dst0, dst1, src0, src1, ex0, ex1, vr0, vr1, dsc0, dsc1,
                out_sh,
                semi0, semi1, semv0, semv1, sems0, sems1):
    c = lax.axis_index("c")
    s = lax.axis_index("s")
    wid = s * NC + c
    base = wid * CE
    vrows = vr0

    def vz_body(r, _):
        for j in range(D // 16):
            vrows[r, pl.ds(j * 16, 16)] = jnp.zeros((16,), jnp.float32)
        return 0

    lax.fori_loop(0, G, vz_body, 0)
    row0 = s * RH
    for i in range(RH // G):
        pltpu.sync_copy(vrows, out_sh.at[pl.ds(row0 + i * G, G)])
    rem = RH % G
    pltpu.sync_copy(
        vrows.at[pl.ds(0, rem)],
        out_sh.at[pl.ds(row0 + (RH // G) * G, rem)],
    )

    @pl.when(s == NS - 1)
    def _zero_tail():
        pltpu.sync_copy(
            vrows.at[pl.ds(0, N - NS * RH)],
            out_sh.at[pl.ds(NS * RH, N - NS * RH)],
        )

    plsc.subcore_barrier()

    bufs = (
        dict(dst=dst0, src=src0, ex=ex0, vr=vr0, dsc=dsc0,
             semi=semi0, semv=semv0, sems=sems0),
        dict(dst=dst1, src=src1, ex=ex1, vr=vr1, dsc=dsc1,
             semi=semi1, semv=semv1, sems=sems1),
    )
    def wait_scat(B):
        pltpu.make_async_copy(
            B["vr"], out_sh.at[B["dsc"]], B["sems"]
        ).wait()

    def fire_idx(b, B):
        off = base + b * G
        pltpu.async_copy(dst_hbm.at[pl.ds(off, G)], B["dst"], B["semi"])
        pltpu.async_copy(src_hbm.at[pl.ds(off, G)], B["src"], B["semi"])
        pltpu.async_copy(ex_hbm.at[pl.ds(off, G)], B["ex"], B["semi"])

    def wait_idx(B):
        zi = dst_hbm.at[pl.ds(0, G)]
        zf = ex_hbm.at[pl.ds(0, G)]
        pltpu.make_async_copy(zi, B["dst"], B["semi"]).wait()
        pltpu.make_async_copy(zi, B["src"], B["semi"]).wait()
        pltpu.make_async_copy(zf, B["ex"], B["semi"]).wait()

    def fire_v(B):
        pltpu.async_copy(v_hbm.at[B["src"]], B["vr"], B["semv"])

    def wait_v(B):
        pltpu.make_async_copy(v_hbm.at[B["src"]], B["vr"], B["semv"]).wait()

    def compute(B):
        dstb, exb, vr, dsc = B["dst"], B["ex"], B["vr"], B["dsc"]

        def cp_body(g, _):
            ix = pl.ds(g * 16, 16)
            dsc[ix] = dstb[ix]
            return 0

        lax.fori_loop(0, LG, cp_body, 0)

        def scale_body(g, _):
            gbase = g * 16
            for i in range(16):
                e = gbase + i
                wv = plsc.load_gather(exb, [jnp.full((16,), e, jnp.int32)])
                for j in range(D // 16):
                    ix = pl.ds(j * 16, 16)
                    vr[e, ix] = vr[e, ix] * wv
            return 0

        lax.fori_loop(0, LG, scale_body, 0)
        pltpu.async_copy(vr, out_sh.at[dsc], B["sems"], add=True)

    fire_idx(0, bufs[0])
    wait_idx(bufs[0])
    fire_v(bufs[0])
    fire_idx(1, bufs[1])

    def pair_body(t, _):
        b0 = 2 * t
        wait_v(bufs[0])
        wait_idx(bufs[1])

        @pl.when(b0 >= 2)
        def _():
            wait_scat(bufs[1])

        fire_v(bufs[1])
        compute(bufs[0])
        fire_idx(b0 + 2, bufs[0])

        wait_v(bufs[1])
        wait_idx(bufs[0])
        wait_scat(bufs[0])
        fire_v(bufs[0])
        compute(bufs[1])

        @pl.when(b0 + 3 < NB)
        def _():
            fire_idx(b0 + 3, bufs[1])

        return 0

    lax.fori_loop(0, (NB - 1) // 2, pair_body, 0)
    wait_v(bufs[0])
    compute(bufs[0])
    wait_scat(bufs[0])
    wait_scat(bufs[1])
    plsc.subcore_barrier()

    rem = RH % G
    for i in range(RH // G):
        r0 = s * RH + i * G
        pltpu.sync_copy(out_sh.at[pl.ds(r0, G)], vrows)
        pltpu.sync_copy(vrows, outp_hbm.at[c, pl.ds(r0, G)])
    r0 = s * RH + (RH // G) * G
    pltpu.sync_copy(out_sh.at[pl.ds(r0, rem)], vrows.at[pl.ds(0, rem)])
    pltpu.sync_copy(vrows.at[pl.ds(0, rem)], outp_hbm.at[c, pl.ds(r0, rem)])

    @pl.when(s == NS - 1)
    def _write_tail():
        nt_ = N - NS * RH
        pltpu.sync_copy(
            out_sh.at[pl.ds(NS * RH, nt_)], vrows.at[pl.ds(0, nt_)]
        )
        pltpu.sync_copy(
            vrows.at[pl.ds(0, nt_)], outp_hbm.at[c, pl.ds(NS * RH, nt_)]
        )


BN2 = 1024


def _norm_body(p_ref, ss_ref, o_ref):
    s2 = jnp.sum(ss_ref[...], axis=0).reshape(BN2, 1)
    o_ref[...] = (p_ref[0] + p_ref[1]) / (s2 + 1e-10)


def _normalize(outp, ssump):
    return pl.pallas_call(
        _norm_body,
        grid=(NPAD // BN2,),
        in_specs=[
            pl.BlockSpec((NC, BN2, D), lambda i: (0, i, 0)),
            pl.BlockSpec((NW, BN2), lambda i: (0, i)),
        ],
        out_specs=pl.BlockSpec((BN2, D), lambda i: (i, 0)),
        out_shape=jax.ShapeDtypeStruct((NPAD, D), jnp.float32),
    )(outp, ssump)


def kernel(x, edge_index, edge_type, node_type, W_Q, W_K, W_V, W_edge, mu):
    src = edge_index[0]
    dst = edge_index[1]
    nt2d = node_type.reshape(N, 1)
    musc = (mu / math.sqrt(D)).astype(jnp.float32)
    q, v, ke = _qvke(x, nt2d, W_Q, W_K, W_V, W_edge, musc)
    ke2 = ke.reshape(ET * N, D)
    ex, ssump = _scores_kernel(q, ke2, src, dst, edge_type)
    outp = _agg_kernel(v, src, dst, ex)
    return _normalize(outp, ssump.reshape(NW, NPAD))[:N]
